# Initial kernel scaffold; baseline (speedup 1.0000x reference)
import jax
import jax.numpy as jnp
from jax.experimental import pallas as pl


def kernel(events):
    raise NotImplementedError("write your pallas kernel here")


# speedup vs baseline: 1.2741x; 1.2741x over previous
import functools

import jax
import jax.numpy as jnp
from jax import lax
from jax.experimental import pallas as pl
from jax.experimental.pallas import tpu as pltpu
from jax.experimental.pallas import tpu_sc as plsc

_VG_W, _VG_H, _VG_T = 32, 24, 5
_XY_SCALE = 0.05
_NBINS = 2 * _VG_T * _VG_H * _VG_W
_NLANE = 16
_NTILES = 32

_N = 4_000_000
_PER_TILE = _N // _NTILES
_CHUNK = 960
_GROUPS = _CHUNK // 16
_NFULL = _PER_TILE // _CHUNK
_TAIL = _PER_TILE - _NFULL * _CHUNK
_CHUNK_W = _CHUNK * 4


def _dma_start(src, dst, sem):
    pltpu.make_async_copy(src, dst, sem).start()


def _dma_wait(src, dst, sem):
    pltpu.make_async_copy(src, dst, sem).wait()


def _mm_body(x_ref, min_ref, max_ref, accmin, accmax):
    i = pl.program_id(0)

    @pl.when(i == 0)
    def _():
        accmin[...] = jnp.full((1, 128), jnp.inf, jnp.float32)
        accmax[...] = jnp.full((1, 128), -jnp.inf, jnp.float32)

    x = x_ref[...]
    lane = lax.broadcasted_iota(jnp.int32, x.shape, 1)
    sel = (lane & 3) == 2
    accmin[...] = jnp.minimum(
        accmin[...], jnp.min(jnp.where(sel, x, jnp.inf), axis=0, keepdims=True))
    accmax[...] = jnp.maximum(
        accmax[...], jnp.max(jnp.where(sel, x, -jnp.inf), axis=0, keepdims=True))

    @pl.when(i == pl.num_programs(0) - 1)
    def _():
        min_ref[0, 0] = jnp.min(accmin[...])
        max_ref[0, 0] = jnp.max(accmax[...])


def _tc_minmax(ev2d):
    rows = ev2d.shape[0]
    br = 1000
    return pl.pallas_call(
        _mm_body,
        grid=(rows // br,),
        in_specs=[pl.BlockSpec((br, 128), lambda i: (i, 0))],
        out_specs=[
            pl.BlockSpec(memory_space=pltpu.SMEM),
            pl.BlockSpec(memory_space=pltpu.SMEM),
        ],
        out_shape=[
            jax.ShapeDtypeStruct((1, 1), jnp.float32),
            jax.ShapeDtypeStruct((1, 1), jnp.float32),
        ],
        scratch_shapes=[
            pltpu.VMEM((1, 128), jnp.float32),
            pltpu.VMEM((1, 128), jnp.float32),
        ],
    )(ev2d)


_sc_mesh = plsc.VectorSubcoreMesh(core_axis_name="c", subcore_axis_name="s")


@functools.partial(
    pl.kernel,
    mesh=_sc_mesh,
    compiler_params=pltpu.CompilerParams(needs_layout_passes=False),
    out_type=jax.ShapeDtypeStruct((_NTILES, _NBINS), jnp.float32),
    scratch_types=[
        pltpu.VMEM((_NLANE * _NBINS,), jnp.float32),
        pltpu.VMEM((_CHUNK_W,), jnp.float32),
        pltpu.VMEM((_CHUNK_W,), jnp.float32),
        pltpu.VMEM((16,), jnp.float32),
        pltpu.VMEM((16,), jnp.float32),
        pltpu.SemaphoreType.DMA,
        pltpu.SemaphoreType.DMA,
    ],
)
def _sc_hist(ev_hbm, toff_hbm, tscl_hbm, out_hbm,
             hist, bufa, bufb, toffv, tsclv, sema, semb):
    wid = lax.axis_index("s") * 2 + lax.axis_index("c")
    base = wid * (_PER_TILE * 4)

    ii = lax.iota(jnp.int32, 16)
    z16 = jnp.zeros((16,), jnp.float32)
    ones = jnp.ones((16,), jnp.float32)
    lane_off = ii * _NBINS

    _dma_start(ev_hbm.at[pl.ds(base, _CHUNK_W)], bufa, sema)
    _dma_start(ev_hbm.at[pl.ds(base + _CHUNK_W, _CHUNK_W)], bufb, semb)
    pltpu.sync_copy(toff_hbm, toffv)
    pltpu.sync_copy(tscl_hbm, tsclv)
    toff = toffv[...]
    tscl = tsclv[...]

    def _zero(i, c):
        b = i * 256
        for k in range(16):
            hist[pl.ds(b + k * 16, 16)] = z16
        return c

    lax.fori_loop(0, (_NLANE * _NBINS) // 256, _zero, 0)

    def _bins(buf, off):
        x = plsc.load_gather(buf, [off])
        y = plsc.load_gather(buf, [off + 1])
        t = plsc.load_gather(buf, [off + 2])
        p = plsc.load_gather(buf, [off + 3])
        xv = jnp.clip((x * _XY_SCALE).astype(jnp.int32), 0, _VG_W - 1)
        yv = jnp.clip((y * _XY_SCALE).astype(jnp.int32), 0, _VG_H - 1)
        tv = jnp.clip(((t - toff) * tscl).astype(jnp.int32), 0, _VG_T - 1)
        ch = jnp.where(p > 0.0, 0, _NBINS // 2)
        return ch + tv * (_VG_H * _VG_W) + yv * _VG_W + xv + lane_off

    def _proc(buf):
        def body(j, c):
            b = _bins(buf, j * 64 + ii * 4)
            plsc.addupdate_scatter(hist, [b], ones)
            return c
        lax.fori_loop(0, _GROUPS, body, 0)

    def _main(i, c):
        c0 = 2 * i
        _dma_wait(ev_hbm.at[pl.ds(base, _CHUNK_W)], bufa, sema)
        _proc(bufa)

        @pl.when(c0 + 2 < _NFULL)
        def _():
            _dma_start(ev_hbm.at[pl.ds(base + (c0 + 2) * _CHUNK_W, _CHUNK_W)],
                       bufa, sema)

        _dma_wait(ev_hbm.at[pl.ds(base, _CHUNK_W)], bufb, semb)
        _proc(bufb)

        @pl.when(c0 + 3 < _NFULL)
        def _():
            _dma_start(ev_hbm.at[pl.ds(base + (c0 + 3) * _CHUNK_W, _CHUNK_W)],
                       bufb, semb)

        return c

    lax.fori_loop(0, _NFULL // 2, _main, 0)

    if _TAIL:
        tw = _TAIL * 4
        pltpu.sync_copy(ev_hbm.at[pl.ds(base + _NFULL * _CHUNK_W, tw)],
                        bufa.at[pl.ds(0, tw)])

        def tbody(j, c):
            b = _bins(bufa, j * 64 + ii * 4)
            plsc.addupdate_scatter(hist, [b], ones)
            return c

        lax.fori_loop(0, _TAIL // 16, tbody, 0)
        rem = _TAIL % 16
        if rem:
            boff = (_TAIL // 16) * 64
            b = _bins(bufa, boff + jnp.minimum(ii, rem - 1) * 4)
            plsc.addupdate_scatter(hist, [b], ones, mask=ii < rem)

    def _reduce(j, c):
        o = j * 16
        s = hist[pl.ds(o, 16)]
        for l in range(1, _NLANE):
            s = s + hist[pl.ds(l * _NBINS + o, 16)]
        hist[pl.ds(o, 16)] = s
        return c

    lax.fori_loop(0, _NBINS // 16, _reduce, 0)
    pltpu.sync_copy(hist.at[pl.ds(0, _NBINS)], out_hbm.at[wid])


def _fin_body(h_ref, o_ref):
    h = h_ref[...]
    s = jnp.sum(h, axis=0, keepdims=True)
    tot = jnp.sum(s)
    o_ref[...] = jnp.where(tot > 0.0, s / tot, s)


def _tc_finalize(parts):
    return pl.pallas_call(
        _fin_body,
        out_shape=jax.ShapeDtypeStruct((1, _NBINS), jnp.float32),
    )(parts)


def kernel(events):
    ev_flat = events.reshape(-1)
    ev2d = events.reshape(-1, 128)
    tmin2, tmax2 = _tc_minmax(ev2d)
    tmin = tmin2[0, 0]
    tmax = tmax2[0, 0]
    cond = tmax > tmin
    denom = jnp.where(cond, tmax - tmin, jnp.float32(1.0))
    t_scl = jnp.where(cond, jnp.float32(_VG_T) / denom, jnp.float32(0.1))
    t_off = jnp.where(cond, tmin, jnp.float32(0.0))
    toff_v = jnp.full((16,), t_off, jnp.float32)
    tscl_v = jnp.full((16,), t_scl, jnp.float32)
    parts = _sc_hist(ev_flat, toff_v, tscl_v)
    flat = _tc_finalize(parts)
    return flat.reshape(2, _VG_T, _VG_H, _VG_W)


# speedup vs baseline: 34.6289x; 27.1781x over previous
import functools

import jax
import jax.numpy as jnp
from jax import lax
from jax.experimental import pallas as pl
from jax.experimental.pallas import tpu as pltpu
from jax.experimental.pallas import tpu_sc as plsc

_VG_W, _VG_H, _VG_T = 32, 24, 5
_XY_SCALE = 0.05
_NBINS = 2 * _VG_T * _VG_H * _VG_W
_NLANE = 16
_NTILES = 32

_N = 4_000_000
_NROWS = _N // 128

_HROWS = 7
_HFULL = 976 // _HROWS

_MROWS = 72
_MCH = 14

_SC_PARAMS = pltpu.CompilerParams(
    needs_layout_passes=False, use_tc_tiling_on_sc=False)
_sc_mesh = plsc.VectorSubcoreMesh(core_axis_name="c", subcore_axis_name="s")


def _dma_start(src, dst, sem):
    pltpu.make_async_copy(src, dst, sem).start()


def _dma_wait(src, dst, sem):
    pltpu.make_async_copy(src, dst, sem).wait()


@functools.partial(
    pl.kernel,
    mesh=_sc_mesh,
    compiler_params=_SC_PARAMS,
    out_type=(
        jax.ShapeDtypeStruct((_NTILES, 16), jnp.float32),
        jax.ShapeDtypeStruct((_NTILES, 16), jnp.float32),
    ),
    scratch_types=[
        pltpu.VMEM((_MROWS, 1, 128), jnp.float32),
        pltpu.VMEM((_MROWS, 1, 128), jnp.float32),
        pltpu.VMEM((16,), jnp.float32),
        pltpu.VMEM((16,), jnp.float32),
        pltpu.SemaphoreType.DMA,
        pltpu.SemaphoreType.DMA,
    ],
)
def _sc_minmax(ev_hbm, min_hbm, max_hbm, bufa, bufb, minv, maxv, sema, semb):
    wid = lax.axis_index("s") * 2 + lax.axis_index("c")
    s = 976 * wid

    def _st(c):
        return jnp.minimum(s + c * _MROWS, _NROWS - _MROWS)

    _dma_start(ev_hbm.at[pl.ds(_st(0), _MROWS), pl.ds(2, 1), :], bufa, sema)
    _dma_start(ev_hbm.at[pl.ds(_st(1), _MROWS), pl.ds(2, 1), :], bufb, semb)

    def _scan(buf, mn, mx):
        def rbody(j, c):
            mn, mx = c
            for a in range(8):
                t = buf[j, 0, pl.ds(a * 16, 16)]
                mn = jnp.minimum(mn, t)
                mx = jnp.maximum(mx, t)
            return mn, mx

        return lax.fori_loop(0, _MROWS, rbody, (mn, mx))

    def _main(i, c):
        mn, mx = c
        c0 = 2 * i
        _dma_wait(ev_hbm.at[pl.ds(0, _MROWS), pl.ds(2, 1), :], bufa, sema)
        mn, mx = _scan(bufa, mn, mx)

        @pl.when(c0 + 2 < _MCH)
        def _():
            _dma_start(ev_hbm.at[pl.ds(_st(c0 + 2), _MROWS), pl.ds(2, 1), :],
                       bufa, sema)

        _dma_wait(ev_hbm.at[pl.ds(0, _MROWS), pl.ds(2, 1), :], bufb, semb)
        mn, mx = _scan(bufb, mn, mx)

        @pl.when(c0 + 3 < _MCH)
        def _():
            _dma_start(ev_hbm.at[pl.ds(_st(c0 + 3), _MROWS), pl.ds(2, 1), :],
                       bufb, semb)

        return mn, mx

    inf = jnp.full((16,), jnp.inf, jnp.float32)
    mn, mx = lax.fori_loop(0, _MCH // 2, _main, (inf, -inf))
    minv[...] = mn
    maxv[...] = mx
    pltpu.sync_copy(minv, min_hbm.at[wid])
    pltpu.sync_copy(maxv, max_hbm.at[wid])


@functools.partial(
    pl.kernel,
    mesh=_sc_mesh,
    compiler_params=_SC_PARAMS,
    out_type=jax.ShapeDtypeStruct((_NTILES, _NBINS), jnp.float32),
    scratch_types=[
        pltpu.VMEM((_NLANE * _NBINS,), jnp.float32),
        pltpu.VMEM((_HROWS, 4, 128), jnp.float32),
        pltpu.VMEM((_HROWS, 4, 128), jnp.float32),
        pltpu.VMEM((16,), jnp.float32),
        pltpu.VMEM((16,), jnp.float32),
        pltpu.SemaphoreType.DMA,
        pltpu.SemaphoreType.DMA,
    ],
)
def _sc_hist(ev_hbm, toff_hbm, tscl_hbm, out_hbm,
             hist, bufa, bufb, toffv, tsclv, sema, semb):
    wid = lax.axis_index("s") * 2 + lax.axis_index("c")
    s = 976 * wid + jnp.minimum(wid, 18)
    n = jnp.where(wid < 18, 977, 976)

    ii = lax.iota(jnp.int32, 16)
    z16 = jnp.zeros((16,), jnp.float32)
    ones = jnp.ones((16,), jnp.float32)
    lane_off = ii * _NBINS

    _dma_start(ev_hbm.at[pl.ds(s, _HROWS), :, :], bufa, sema)
    _dma_start(ev_hbm.at[pl.ds(s + _HROWS, _HROWS), :, :], bufb, semb)
    pltpu.sync_copy(toff_hbm, toffv)
    pltpu.sync_copy(tscl_hbm, tsclv)
    toff = toffv[...]
    tscl = tsclv[...]

    def _zero(i, c):
        b = i * 256
        for k in range(16):
            hist[pl.ds(b + k * 16, 16)] = z16
        return c

    lax.fori_loop(0, (_NLANE * _NBINS) // 256, _zero, 0)

    def _bins(buf, j, a):
        sl = pl.ds(a * 16, 16)
        x = buf[j, 0, sl]
        y = buf[j, 1, sl]
        t = buf[j, 2, sl]
        p = buf[j, 3, sl]
        xv = jnp.clip((x * _XY_SCALE).astype(jnp.int32), 0, _VG_W - 1)
        yv = jnp.clip((y * _XY_SCALE).astype(jnp.int32), 0, _VG_H - 1)
        tv = jnp.clip(((t - toff) * tscl).astype(jnp.int32), 0, _VG_T - 1)
        ch = jnp.where(p > 0.0, 0, _NBINS // 2)
        return ch + tv * (_VG_H * _VG_W) + yv * _VG_W + xv + lane_off

    def _proc(buf):
        def rbody(j, c):
            for a in range(8):
                plsc.addupdate_scatter(hist, [_bins(buf, j, a)], ones)
            return c

        lax.fori_loop(0, _HROWS, rbody, 0)

    def _main(i, c):
        c0 = 2 * i
        _dma_wait(ev_hbm.at[pl.ds(0, _HROWS), :, :], bufa, sema)
        _proc(bufa)

        @pl.when(c0 + 2 < _HFULL)
        def _():
            _dma_start(ev_hbm.at[pl.ds(s + (c0 + 2) * _HROWS, _HROWS), :, :],
                       bufa, sema)

        _dma_wait(ev_hbm.at[pl.ds(0, _HROWS), :, :], bufb, semb)
        _proc(bufb)

        @pl.when(c0 + 3 < _HFULL)
        def _():
            _dma_start(ev_hbm.at[pl.ds(s + (c0 + 3) * _HROWS, _HROWS), :, :],
                       bufb, semb)

        return c

    lax.fori_loop(0, (_HFULL - 1) // 2, _main, 0)
    _dma_wait(ev_hbm.at[pl.ds(0, _HROWS), :, :], bufa, sema)
    _proc(bufa)

    pltpu.sync_copy(ev_hbm.at[pl.ds(s + n - 4, 4), :, :],
                    bufb.at[pl.ds(0, 4), :, :])
    for r in range(4):
        keep = (jnp.full((16,), r, jnp.int32) + n) >= 977
        for a in range(8):
            plsc.addupdate_scatter(hist, [_bins(bufb, r, a)], ones, mask=keep)

    def _reduce(j, c):
        o = j * 16
        acc = hist[pl.ds(o, 16)]
        for l in range(1, _NLANE):
            acc = acc + hist[pl.ds(l * _NBINS + o, 16)]
        hist[pl.ds(o, 16)] = acc
        return c

    lax.fori_loop(0, _NBINS // 16, _reduce, 0)
    pltpu.sync_copy(hist.at[pl.ds(0, _NBINS)], out_hbm.at[wid])


def _fin_body(h_ref, o_ref):
    h = h_ref[...]
    t = jnp.sum(h, axis=0, keepdims=True)
    tot = jnp.sum(t)
    o_ref[...] = jnp.where(tot > 0.0, t / tot, t)


def _tc_finalize(parts):
    return pl.pallas_call(
        _fin_body,
        out_shape=jax.ShapeDtypeStruct((1, _NBINS), jnp.float32),
    )(parts)


def kernel(events):
    ev3 = events.reshape(_NROWS, 128, 4).transpose(0, 2, 1)
    mins, maxs = _sc_minmax(ev3)
    tmin = jnp.min(mins)
    tmax = jnp.max(maxs)
    cond = tmax > tmin
    denom = jnp.where(cond, tmax - tmin, jnp.float32(1.0))
    t_scl = jnp.where(cond, jnp.float32(_VG_T) / denom, jnp.float32(0.1))
    t_off = jnp.where(cond, tmin, jnp.float32(0.0))
    toff_v = jnp.full((16,), t_off, jnp.float32)
    tscl_v = jnp.full((16,), t_scl, jnp.float32)
    parts = _sc_hist(ev3, toff_v, tscl_v)
    flat = _tc_finalize(parts)
    return flat.reshape(2, _VG_T, _VG_H, _VG_W)


# speedup vs baseline: 66.1635x; 1.9106x over previous
import functools

import jax
import jax.numpy as jnp
from jax import lax
from jax.experimental import pallas as pl
from jax.experimental.pallas import tpu as pltpu
from jax.experimental.pallas import tpu_sc as plsc

_VG_W, _VG_H, _VG_T = 32, 24, 5
_XY_SCALE = 0.05
_NBINS = 2 * _VG_T * _VG_H * _VG_W
_NLANE = 16
_NTILES = 32

_N = 4_000_000
_NROWS = _N // 128

_HROWS = 7
_HFULL = 976 // _HROWS

_MROWS = 72
_MCH = 14

_SKEW = _NBINS + 1
_HALLOC = 481 * 256

_SC_PARAMS = pltpu.CompilerParams(
    needs_layout_passes=False, use_tc_tiling_on_sc=False)
_sc_mesh = plsc.VectorSubcoreMesh(core_axis_name="c", subcore_axis_name="s")


def _dma_start(src, dst, sem):
    pltpu.make_async_copy(src, dst, sem).start()


def _dma_wait(src, dst, sem):
    pltpu.make_async_copy(src, dst, sem).wait()


@functools.partial(
    pl.kernel,
    mesh=_sc_mesh,
    compiler_params=_SC_PARAMS,
    out_type=(
        jax.ShapeDtypeStruct((_NTILES, 16), jnp.float32),
        jax.ShapeDtypeStruct((_NTILES, 16), jnp.float32),
    ),
    scratch_types=[
        pltpu.VMEM((_MROWS, 1, 128), jnp.float32),
        pltpu.VMEM((_MROWS, 1, 128), jnp.float32),
        pltpu.VMEM((16,), jnp.float32),
        pltpu.VMEM((16,), jnp.float32),
        pltpu.SemaphoreType.DMA,
        pltpu.SemaphoreType.DMA,
    ],
)
def _sc_minmax(ev_hbm, min_hbm, max_hbm, bufa, bufb, minv, maxv, sema, semb):
    wid = lax.axis_index("s") * 2 + lax.axis_index("c")
    s = 976 * wid

    def _st(c):
        return jnp.minimum(s + c * _MROWS, _NROWS - _MROWS)

    _dma_start(ev_hbm.at[pl.ds(_st(0), _MROWS), pl.ds(2, 1), :], bufa, sema)
    _dma_start(ev_hbm.at[pl.ds(_st(1), _MROWS), pl.ds(2, 1), :], bufb, semb)

    def _scan(buf, mn, mx):
        def rbody(j, c):
            mn, mx = c
            for a in range(8):
                t = buf[j, 0, pl.ds(a * 16, 16)]
                mn = jnp.minimum(mn, t)
                mx = jnp.maximum(mx, t)
            return mn, mx

        return lax.fori_loop(0, _MROWS, rbody, (mn, mx))

    def _main(i, c):
        mn, mx = c
        c0 = 2 * i
        _dma_wait(ev_hbm.at[pl.ds(0, _MROWS), pl.ds(2, 1), :], bufa, sema)
        mn, mx = _scan(bufa, mn, mx)

        @pl.when(c0 + 2 < _MCH)
        def _():
            _dma_start(ev_hbm.at[pl.ds(_st(c0 + 2), _MROWS), pl.ds(2, 1), :],
                       bufa, sema)

        _dma_wait(ev_hbm.at[pl.ds(0, _MROWS), pl.ds(2, 1), :], bufb, semb)
        mn, mx = _scan(bufb, mn, mx)

        @pl.when(c0 + 3 < _MCH)
        def _():
            _dma_start(ev_hbm.at[pl.ds(_st(c0 + 3), _MROWS), pl.ds(2, 1), :],
                       bufb, semb)

        return mn, mx

    inf = jnp.full((16,), jnp.inf, jnp.float32)
    mn, mx = lax.fori_loop(0, _MCH // 2, _main, (inf, -inf))
    minv[...] = mn
    maxv[...] = mx
    pltpu.sync_copy(minv, min_hbm.at[wid])
    pltpu.sync_copy(maxv, max_hbm.at[wid])


@functools.partial(
    pl.kernel,
    mesh=_sc_mesh,
    compiler_params=_SC_PARAMS,
    out_type=jax.ShapeDtypeStruct((_NTILES, _NBINS), jnp.float32),
    scratch_types=[
        pltpu.VMEM((_HALLOC,), jnp.float32),
        pltpu.VMEM((_HROWS, 4, 128), jnp.float32),
        pltpu.VMEM((_HROWS, 4, 128), jnp.float32),
        pltpu.VMEM((16,), jnp.float32),
        pltpu.VMEM((16,), jnp.float32),
        pltpu.SemaphoreType.DMA,
        pltpu.SemaphoreType.DMA,
    ],
)
def _sc_hist(ev_hbm, toff_hbm, tscl_hbm, out_hbm,
             hist, bufa, bufb, toffv, tsclv, sema, semb):
    wid = lax.axis_index("s") * 2 + lax.axis_index("c")
    s = 976 * wid + jnp.minimum(wid, 18)
    n = jnp.where(wid < 18, 977, 976)

    ii = lax.iota(jnp.int32, 16)
    z16 = jnp.zeros((16,), jnp.float32)
    ones = jnp.ones((16,), jnp.float32)
    lane_off = ii * _SKEW

    _dma_start(ev_hbm.at[pl.ds(s, _HROWS), :, :], bufa, sema)
    _dma_start(ev_hbm.at[pl.ds(s + _HROWS, _HROWS), :, :], bufb, semb)
    pltpu.sync_copy(toff_hbm, toffv)
    pltpu.sync_copy(tscl_hbm, tsclv)
    toff = toffv[...]
    tscl = tsclv[...]

    def _zero(i, c):
        b = i * 256
        for k in range(16):
            hist[pl.ds(b + k * 16, 16)] = z16
        return c

    lax.fori_loop(0, _HALLOC // 256, _zero, 0)

    def _bins(buf, j, a):
        sl = pl.ds(a * 16, 16)
        x = buf[j, 0, sl]
        y = buf[j, 1, sl]
        t = buf[j, 2, sl]
        p = buf[j, 3, sl]
        xv = jnp.clip((x * _XY_SCALE).astype(jnp.int32), 0, _VG_W - 1)
        yv = jnp.clip((y * _XY_SCALE).astype(jnp.int32), 0, _VG_H - 1)
        tv = jnp.clip(((t - toff) * tscl).astype(jnp.int32), 0, _VG_T - 1)
        ch = jnp.where(p > 0.0, 0, _NBINS // 2)
        return ch + tv * (_VG_H * _VG_W) + yv * _VG_W + xv + lane_off

    def _proc(buf):
        def rbody(j, c):
            bs = [_bins(buf, j, a) for a in range(8)]
            for b in bs:
                plsc.addupdate_scatter(hist, [b], ones)
            return c

        lax.fori_loop(0, _HROWS, rbody, 0)

    def _main(i, c):
        c0 = 2 * i
        _dma_wait(ev_hbm.at[pl.ds(0, _HROWS), :, :], bufa, sema)
        _proc(bufa)

        @pl.when(c0 + 2 < _HFULL)
        def _():
            _dma_start(ev_hbm.at[pl.ds(s + (c0 + 2) * _HROWS, _HROWS), :, :],
                       bufa, sema)

        _dma_wait(ev_hbm.at[pl.ds(0, _HROWS), :, :], bufb, semb)
        _proc(bufb)

        @pl.when(c0 + 3 < _HFULL)
        def _():
            _dma_start(ev_hbm.at[pl.ds(s + (c0 + 3) * _HROWS, _HROWS), :, :],
                       bufb, semb)

        return c

    lax.fori_loop(0, (_HFULL - 1) // 2, _main, 0)
    _dma_wait(ev_hbm.at[pl.ds(0, _HROWS), :, :], bufa, sema)
    _proc(bufa)

    pltpu.sync_copy(ev_hbm.at[pl.ds(s + n - 4, 4), :, :],
                    bufb.at[pl.ds(0, 4), :, :])
    for r in range(4):
        keep = (jnp.full((16,), r, jnp.int32) + n) >= 977
        for a in range(8):
            plsc.addupdate_scatter(hist, [_bins(bufb, r, a)], ones, mask=keep)

    def _reduce(j, c):
        o = j * 16
        acc = hist[pl.ds(o, 16)]
        for l in range(1, _NLANE):
            acc = acc + hist[pl.ds(l * _SKEW + o, 16)]
        hist[pl.ds(o, 16)] = acc
        return c

    lax.fori_loop(0, _NBINS // 16, _reduce, 0)
    pltpu.sync_copy(hist.at[pl.ds(0, _NBINS)], out_hbm.at[wid])


def _fin_body(h_ref, o_ref):
    h = h_ref[...]
    t = jnp.sum(h, axis=0, keepdims=True)
    tot = jnp.sum(t)
    o_ref[...] = jnp.where(tot > 0.0, t / tot, t)


def _tc_finalize(parts):
    return pl.pallas_call(
        _fin_body,
        out_shape=jax.ShapeDtypeStruct((1, _NBINS), jnp.float32),
    )(parts)


def kernel(events):
    ev3 = events.reshape(_NROWS, 128, 4).transpose(0, 2, 1)
    mins, maxs = _sc_minmax(ev3)
    tmin = jnp.min(mins)
    tmax = jnp.max(maxs)
    cond = tmax > tmin
    denom = jnp.where(cond, tmax - tmin, jnp.float32(1.0))
    t_scl = jnp.where(cond, jnp.float32(_VG_T) / denom, jnp.float32(0.1))
    t_off = jnp.where(cond, tmin, jnp.float32(0.0))
    toff_v = jnp.full((16,), t_off, jnp.float32)
    tscl_v = jnp.full((16,), t_scl, jnp.float32)
    parts = _sc_hist(ev3, toff_v, tscl_v)
    flat = _tc_finalize(parts)
    return flat.reshape(2, _VG_T, _VG_H, _VG_W)
